# baseline (speedup 1.0000x reference)
import functools

import jax
import jax.numpy as jnp
from jax import lax
from jax.experimental import pallas as pl
from jax.experimental.pallas import tpu as pltpu
from jax.experimental.pallas import tpu_sc as plsc

_C = 96
_V = 512 * 512
_B = 1048576
_NC = 2
_NS = 16
_NW = _NC * _NS
_BW = _B // _NW
_CHUNK = 128
_NCH = _BW // _CHUNK
_NBUF = 4

_BH = 16
_GRID_T = 512 // _BH


def _transpose_body(img_ref, out_ref):
    x = img_ref[0].reshape(_C, _BH * 512)
    out_ref[...] = x.T


_tc_transpose = pl.pallas_call(
    _transpose_body,
    grid=(_GRID_T,),
    in_specs=[pl.BlockSpec((1, _C, _BH, 512), lambda i: (0, 0, i, 0))],
    out_specs=pl.BlockSpec((_BH * 512, _C), lambda i: (i, 0)),
    out_shape=jax.ShapeDtypeStruct((_V, _C), jnp.float32),
)


def _gather_body(table_hbm, idx_hbm, out_hbm, idx_v, rows_v, *sems):
    wid = lax.axis_index("s") * _NC + lax.axis_index("c")
    pltpu.sync_copy(idx_hbm.at[pl.ds(wid * _NCH, _NCH)], idx_v)
    out_base = wid * _BW

    for b in range(_NBUF):
        pltpu.async_copy(table_hbm.at[idx_v.at[b]], rows_v.at[b], sems[b])

    def step(g, carry):
        for b in range(_NBUF):
            j = g * _NBUF + b
            pltpu.make_async_copy(
                table_hbm.at[idx_v.at[j]], rows_v.at[b], sems[b]).wait()
            pltpu.sync_copy(
                rows_v.at[b],
                out_hbm.at[pl.ds(out_base + j * _CHUNK, _CHUNK)])

            @pl.when(j + _NBUF < _NCH)
            def _():
                pltpu.async_copy(
                    table_hbm.at[idx_v.at[j + _NBUF]], rows_v.at[b], sems[b])
        return carry

    lax.fori_loop(0, _NCH // _NBUF, step, 0)


_sc_gather = functools.partial(
    pl.kernel,
    out_type=jax.ShapeDtypeStruct((_B, _C), jnp.float32),
    mesh=plsc.VectorSubcoreMesh(core_axis_name="c", subcore_axis_name="s"),
    scratch_types=[
        pltpu.VMEM((_NCH, _CHUNK), jnp.int32),
        pltpu.VMEM((_NBUF, _CHUNK, _C), jnp.float32),
    ] + [pltpu.SemaphoreType.DMA] * _NBUF,
    compiler_params=pltpu.CompilerParams(use_tc_tiling_on_sc=False),
)(_gather_body)


def kernel(img, indices):
    table = _tc_transpose(img)
    idx2 = indices.astype(jnp.int32).reshape(_NW * _NCH, _CHUNK)
    return _sc_gather(table, idx2)


# speedup vs baseline: 1.0716x; 1.0716x over previous
import functools

import jax
import jax.numpy as jnp
from jax import lax
from jax.experimental import pallas as pl
from jax.experimental.pallas import tpu as pltpu
from jax.experimental.pallas import tpu_sc as plsc

_C = 96
_PAD = 128
_V = 512 * 512
_B = 1048576
_NC = 2
_NS = 16
_NW = _NC * _NS
_CHUNK = 128
_NBUF = 4

_HALF = _B // 2
_BWH = _HALF // _NW
_NCHH = _BWH // _CHUNK
_IDXROWS = _B // _CHUNK

_BH = 16
_GRID_T = 512 // _BH


def _transpose_body(img_ref, out_ref):
    x = img_ref[0].reshape(_C, _BH * 512)
    out_ref[:, 0:_C] = x.T


_tc_transpose = pl.pallas_call(
    _transpose_body,
    grid=(_GRID_T,),
    in_specs=[pl.BlockSpec((1, _C, _BH, 512), lambda i: (0, 0, i, 0))],
    out_specs=pl.BlockSpec((_BH * 512, _PAD), lambda i: (i, 0)),
    out_shape=jax.ShapeDtypeStruct((_V, _PAD), jnp.float32),
)


def _gather_body(row0, table_hbm, idx_hbm, out_hbm, idx_v, rows_v, *sems):
    wid = lax.axis_index("s") * _NC + lax.axis_index("c")
    pltpu.sync_copy(idx_hbm.at[pl.ds(row0 + wid * _NCHH, _NCHH)], idx_v)
    out_base = wid * _BWH

    for b in range(_NBUF):
        pltpu.async_copy(table_hbm.at[idx_v.at[b]], rows_v.at[b], sems[b])

    def step(g, carry):
        for b in range(_NBUF):
            j = g * _NBUF + b
            pltpu.make_async_copy(
                table_hbm.at[idx_v.at[j]], rows_v.at[b], sems[b]).wait()
            pltpu.sync_copy(
                rows_v.at[b],
                out_hbm.at[pl.ds(out_base + j * _CHUNK, _CHUNK)])

            @pl.when(j + _NBUF < _NCHH)
            def _():
                pltpu.async_copy(
                    table_hbm.at[idx_v.at[j + _NBUF]], rows_v.at[b], sems[b])
        return carry

    lax.fori_loop(0, _NCHH // _NBUF, step, 0)


def _make_sc_gather(row0):
    return functools.partial(
        pl.kernel,
        out_type=jax.ShapeDtypeStruct((_HALF, _PAD), jnp.float32),
        mesh=plsc.VectorSubcoreMesh(core_axis_name="c", subcore_axis_name="s"),
        scratch_types=[
            pltpu.VMEM((_NCHH, _CHUNK), jnp.int32),
            pltpu.VMEM((_NBUF, _CHUNK, _PAD), jnp.float32),
        ] + [pltpu.SemaphoreType.DMA] * _NBUF,
    )(functools.partial(_gather_body, row0))


_sc_gather_a = _make_sc_gather(0)
_sc_gather_b = _make_sc_gather(_IDXROWS // 2)

_SLICE_BLK = 4096
_GRID_S = _HALF // _SLICE_BLK


def _slice_a_body(pad_ref, out_ref):
    out_ref[...] = pad_ref[:, 0:_C]


_tc_slice_a = pl.pallas_call(
    _slice_a_body,
    grid=(_GRID_S,),
    in_specs=[pl.BlockSpec((_SLICE_BLK, _PAD), lambda i: (i, 0))],
    out_specs=pl.BlockSpec((_SLICE_BLK, _C), lambda i: (i, 0)),
    out_shape=jax.ShapeDtypeStruct((_B, _C), jnp.float32),
)


def _slice_b_body(pad_ref, prev_ref, out_ref):
    out_ref[...] = pad_ref[:, 0:_C]


_tc_slice_b = pl.pallas_call(
    _slice_b_body,
    grid=(_GRID_S,),
    in_specs=[
        pl.BlockSpec((_SLICE_BLK, _PAD), lambda i: (i, 0)),
        pl.BlockSpec(memory_space=pl.ANY),
    ],
    out_specs=pl.BlockSpec((_SLICE_BLK, _C), lambda i: (i + _GRID_S, 0)),
    out_shape=jax.ShapeDtypeStruct((_B, _C), jnp.float32),
    input_output_aliases={1: 0},
)


def kernel(img, indices):
    table = _tc_transpose(img)
    idx2 = indices.astype(jnp.int32).reshape(_IDXROWS, _CHUNK)
    pad_a = _sc_gather_a(table, idx2)
    pad_b = _sc_gather_b(table, idx2)
    out = _tc_slice_a(pad_a)
    out = _tc_slice_b(pad_b, out)
    return out


# speedup vs baseline: 1.6525x; 1.5421x over previous
import functools

import jax
import jax.numpy as jnp
from jax import lax
from jax.experimental import pallas as pl
from jax.experimental.pallas import tpu as pltpu
from jax.experimental.pallas import tpu_sc as plsc

_C = 96
_PAD = 128
_V = 512 * 512
_B = 1048576
_NC = 2
_NS = 16
_NW = _NC * _NS
_BW = _B // _NW
_CHUNK = 128
_NCH = _BW // _CHUNK
_NBUF = 5
_DEPTH = 4

_BH = 16
_GRID_T = 512 // _BH


def _transpose_body(img_ref, out_ref):
    x = img_ref[0].reshape(_C, _BH * 512)
    out_ref[:, 0:_C] = x.T


_tc_transpose = pl.pallas_call(
    _transpose_body,
    grid=(_GRID_T,),
    in_specs=[pl.BlockSpec((1, _C, _BH, 512), lambda i: (0, 0, i, 0))],
    out_specs=pl.BlockSpec((_BH * 512, _PAD), lambda i: (i, 0)),
    out_shape=jax.ShapeDtypeStruct((_V, _PAD), jnp.float32),
)


def _gather_body(table_hbm, idx_hbm, out_hbm, idx_v, rows_v, gsems, osems):
    wid = lax.axis_index("s") * _NC + lax.axis_index("c")
    pltpu.sync_copy(idx_hbm.at[pl.ds(wid * _NCH, _NCH)], idx_v)
    out_base = wid * _BW

    def gather_desc(j, b):
        return pltpu.make_async_copy(
            table_hbm.at[idx_v.at[j]], rows_v.at[b], gsems.at[b])

    def out_desc(j, b):
        return pltpu.make_async_copy(
            rows_v.at[b], out_hbm.at[pl.ds(out_base + j * _CHUNK, _CHUNK)],
            osems.at[b])

    def step(t, carry):
        @pl.when(t < _NCH)
        def _():
            b = t % _NBUF

            @pl.when(t >= _NBUF)
            def _():
                out_desc(t - _NBUF, b).wait()

            gather_desc(t, b).start()

        p = t - _DEPTH

        @pl.when(p >= 0)
        def _():
            bp = p % _NBUF
            gather_desc(p, bp).wait()
            out_desc(p, bp).start()

        return carry

    lax.fori_loop(0, _NCH + _DEPTH, step, 0)

    for b in range(_NBUF):
        j = _NCH - _NBUF + b
        out_desc(j, j % _NBUF).wait()


_sc_gather = functools.partial(
    pl.kernel,
    out_type=jax.ShapeDtypeStruct((_B, _PAD), jnp.float32),
    mesh=plsc.VectorSubcoreMesh(core_axis_name="c", subcore_axis_name="s"),
    scratch_types=[
        pltpu.VMEM((_NCH, _CHUNK), jnp.int32),
        pltpu.VMEM((_NBUF, _CHUNK, _PAD), jnp.float32),
        pltpu.SemaphoreType.DMA((_NBUF,)),
        pltpu.SemaphoreType.DMA((_NBUF,)),
    ],
)(_gather_body)


def kernel(img, indices):
    table = _tc_transpose(img)
    idx2 = indices.astype(jnp.int32).reshape(_NW * _NCH, _CHUNK)
    return _sc_gather(table, idx2)[:, :_C]
